# Initial kernel scaffold; baseline (speedup 1.0000x reference)
import jax
import jax.numpy as jnp
from jax.experimental import pallas as pl


def kernel(x, e, edge_index, Wq, Wk, Wv, We_attn, Wo, bo, Weu, beu, g1, b1, ge_w, be_w, g2, b2, Wf1, bf1, Wf2, bf2):
    raise NotImplementedError("write your pallas kernel here")


# speedup vs baseline: 3.4736x; 3.4736x over previous
import functools
import math

import jax
import jax.numpy as jnp
from jax import lax
from jax.experimental import pallas as pl
from jax.experimental.pallas import tpu as pltpu
from jax.experimental.pallas import tpu_sc as plsc

NC = 2
NS = 16
NW = NC * NS
L = 16
CH = 128


def _tc_proj_body(x_ref, wq_ref, wk_ref, wv_ref, xq_ref, xk_ref, xv_ref):
    xb = x_ref[...]
    xq_ref[...] = jnp.dot(xb, wq_ref[...], preferred_element_type=jnp.float32)
    xk_ref[...] = jnp.dot(xb, wk_ref[...], preferred_element_type=jnp.float32)
    xv_ref[...] = jnp.dot(xb, wv_ref[...], preferred_element_type=jnp.float32)


def _tc_ew_body(e_ref, w_ref, ew_ref):
    ew_ref[...] = jnp.dot(e_ref[...], w_ref[...],
                          preferred_element_type=jnp.float32)


def _layer_norm(v, g, b, eps=1e-5):
    m = jnp.mean(v, axis=-1, keepdims=True)
    var = jnp.mean((v - m) * (v - m), axis=-1, keepdims=True)
    return (v - m) / jnp.sqrt(var + eps) * g + b


def _tc_node_body(o0_ref, o1_ref, d0_ref, d1_ref, p_ref, x_ref, wo_ref,
                  bo_ref, g1_ref, b1_ref,
                  weu1_ref, weu2_ref, beu_ref, wf1_ref, bf1_ref, wf2_ref,
                  bf2_ref, g2_ref, b2_ref, xout_ref, a_ref, bbuf_ref):
    den = jnp.dot(d0_ref[...] + d1_ref[...], p_ref[...],
                  preferred_element_type=jnp.float32) + 1e-9
    agg = (o0_ref[...] + o1_ref[...]) / den
    out = jnp.dot(agg, wo_ref[...],
                  preferred_element_type=jnp.float32) + bo_ref[...]
    x1 = _layer_norm(x_ref[...] + out, g1_ref[...], b1_ref[...])
    a_ref[...] = jnp.dot(x1, weu1_ref[...], preferred_element_type=jnp.float32)
    bbuf_ref[...] = jnp.dot(x1, weu2_ref[...],
                            preferred_element_type=jnp.float32) + beu_ref[...]
    h = jnp.dot(x1, wf1_ref[...], preferred_element_type=jnp.float32) + bf1_ref[...]
    h = 0.5 * h * (1.0 + lax.erf(h * (1.0 / math.sqrt(2.0))))
    h = jnp.dot(h, wf2_ref[...], preferred_element_type=jnp.float32) + bf2_ref[...]
    xout_ref[...] = _layer_norm(x1 + h, g2_ref[...], b2_ref[...])


def _tc_eout_body(e_ref, g_ref, gw_ref, bw_ref, eout_ref):
    eout_ref[...] = _layer_norm(e_ref[...] + g_ref[...], gw_ref[...], bw_ref[...])


def _wid(cid, sid):
    return sid * NC + cid


def _node_split(n_nodes):
    rps8 = (n_nodes // NS) // 8 * 8
    tail = n_nodes - rps8 * NS
    return rps8, tail


def _sliced_copy(src_ref, dst_ref, sid, rps8, tail):
    nfull, rem = rps8 // CH, rps8 % CH

    @pl.loop(0, nfull)
    def _cp(t):
        rs = pl.ds(sid * rps8 + t * CH, CH)
        pltpu.sync_copy(src_ref.at[rs], dst_ref.at[rs])
    if rem:
        rs = pl.ds(sid * rps8 + nfull * CH, rem)
        pltpu.sync_copy(src_ref.at[rs], dst_ref.at[rs])
    if tail:
        @pl.when(sid == NS - 1)
        def _():
            ts = pl.ds(NS * rps8, tail)
            pltpu.sync_copy(src_ref.at[ts], dst_ref.at[ts])


def _zero_fill(zbuf_ref, dst_ref, sid, rps8, tail):
    zlen = zbuf_ref.shape[0]
    nfull, rem = rps8 // zlen, rps8 % zlen

    @pl.loop(0, nfull)
    def _cp(t):
        pltpu.sync_copy(zbuf_ref,
                        dst_ref.at[pl.ds(sid * rps8 + t * zlen, zlen)])
    if rem:
        pltpu.sync_copy(zbuf_ref.at[pl.ds(0, rem)],
                        dst_ref.at[pl.ds(sid * rps8 + nfull * zlen, rem)])
    if tail:
        @pl.when(sid == NS - 1)
        def _():
            pltpu.sync_copy(zbuf_ref.at[pl.ds(0, tail)],
                            dst_ref.at[pl.ds(NS * rps8, tail)])


def _sc_attn_body(nch, ch, n_nodes, xq_hbm, xk_hbm, ew_hbm, src_hbm, dst_hbm,
                  aexp_hbm, d0_hbm, d1_hbm,
                  src_v, dst_v, q_v, k_v, ew_v, aexp_v, pad_v, dsh, sem):
    cid = lax.axis_index("c")
    sid = lax.axis_index("s")
    w = _wid(cid, sid)
    rps8, tail = _node_split(n_nodes)

    @pl.loop(0, ch)
    def _zb(i):
        for h in range(8):
            pad_v[i, pl.ds(h * 16, 16)] = jnp.zeros((16,), jnp.float32)
    _zero_fill(pad_v, dsh, sid, rps8, tail)
    plsc.subcore_barrier()

    lanes = lax.iota(jnp.int32, L)
    n_iters = (nch + NW - 1) // NW

    @pl.loop(0, n_iters)
    def _chunk(j):
        c = w + NW * j

        @pl.when(c < nch)
        def _():
            base = c * ch
            pltpu.sync_copy(src_hbm.at[pl.ds(base, ch)], src_v)
            pltpu.sync_copy(dst_hbm.at[pl.ds(base, ch)], dst_v)
            dq = pltpu.async_copy(xq_hbm.at[dst_v], q_v, sem)
            dk = pltpu.async_copy(xk_hbm.at[src_v], k_v, sem)
            pltpu.sync_copy(ew_hbm.at[pl.ds(base, ch)], ew_v)
            dq.wait()
            dk.wait()

            @pl.loop(0, ch)
            def _edge(i):
                av = jnp.zeros((L,), jnp.float32)
                for h in range(8):
                    qh = q_v[i, pl.ds(h * 16, 16)]
                    kh = k_v[i, pl.ds(h * 16, 16)]
                    s = jnp.sum(qh * kh)
                    av = jnp.where(lanes == h, s, av)
                a = av * 0.25 + ew_v[i, :]
                row = jnp.where(lanes < 8, jnp.exp(a), 0.0)
                aexp_v[i, :] = row
                pad_v[i, pl.ds(0, L)] = row

            pltpu.sync_copy(pad_v, dsh.at[dst_v], add=True)
            pltpu.sync_copy(aexp_v, aexp_hbm.at[pl.ds(base, ch)])

    plsc.subcore_barrier()

    @pl.when(cid == 0)
    def _():
        _sliced_copy(dsh, d0_hbm, sid, rps8, tail)

    @pl.when(cid == 1)
    def _():
        _sliced_copy(dsh, d1_hbm, sid, rps8, tail)


def _sc_agg_body(nch, n_nodes, xv_hbm, aexp_hbm, src_hbm,
                 dst_hbm, o0_hbm, o1_hbm,
                 src_v, dst_v, v_v, sc_v, aexp_v, osh, sem):
    cid = lax.axis_index("c")
    sid = lax.axis_index("s")
    w = _wid(cid, sid)
    rps8, tail = _node_split(n_nodes)

    @pl.loop(0, CH)
    def _zb(i):
        for h in range(8):
            sc_v[i, pl.ds(h * 16, 16)] = jnp.zeros((16,), jnp.float32)
    _zero_fill(sc_v, osh, sid, rps8, tail)
    plsc.subcore_barrier()

    n_iters = (nch + NW - 1) // NW

    @pl.loop(0, n_iters)
    def _chunk(j):
        c = w + NW * j

        @pl.when(c < nch)
        def _():
            base = c * CH
            pltpu.sync_copy(src_hbm.at[pl.ds(base, CH)], src_v)
            pltpu.sync_copy(dst_hbm.at[pl.ds(base, CH)], dst_v)
            dv = pltpu.async_copy(xv_hbm.at[src_v], v_v, sem)
            pltpu.sync_copy(aexp_hbm.at[pl.ds(base, CH)], aexp_v)
            dv.wait()

            @pl.loop(0, CH)
            def _scale(i):
                nr = aexp_v[i, :]
                for h in range(8):
                    s = nr[h]
                    sc_v[i, pl.ds(h * 16, 16)] = v_v[i, pl.ds(h * 16, 16)] * s

            pltpu.sync_copy(sc_v, osh.at[dst_v], add=True)

    plsc.subcore_barrier()

    @pl.when(cid == 0)
    def _():
        _sliced_copy(osh, o0_hbm, sid, rps8, tail)

    @pl.when(cid == 1)
    def _():
        _sliced_copy(osh, o1_hbm, sid, rps8, tail)


def _sc_eg_body(nch, a_hbm, b_hbm, src_hbm, dst_hbm, g_hbm,
                src_v, dst_v, a_v, b_v, sem):
    cid = lax.axis_index("c")
    sid = lax.axis_index("s")
    w = _wid(cid, sid)
    n_iters = (nch + NW - 1) // NW

    @pl.loop(0, n_iters)
    def _chunk(j):
        c = w + NW * j

        @pl.when(c < nch)
        def _():
            base = c * CH
            pltpu.sync_copy(src_hbm.at[pl.ds(base, CH)], src_v)
            pltpu.sync_copy(dst_hbm.at[pl.ds(base, CH)], dst_v)
            da = pltpu.async_copy(a_hbm.at[src_v], a_v, sem)
            db = pltpu.async_copy(b_hbm.at[dst_v], b_v, sem)
            da.wait()
            db.wait()

            @pl.loop(0, CH)
            def _edge(i):
                for h in range(8):
                    sl = pl.ds(h * 16, 16)
                    a_v[i, sl] = a_v[i, sl] + b_v[i, sl]

            pltpu.sync_copy(a_v, g_hbm.at[pl.ds(base, CH)])


def kernel(x, e, edge_index, Wq, Wk, Wv, We_attn, Wo, bo, Weu, beu,
           g1, b1, ge_w, be_w, g2, b2, Wf1, bf1, Wf2, bf2):
    N, D = x.shape
    E = e.shape[0]
    nch = E // CH
    f32 = jnp.float32
    mesh = plsc.VectorSubcoreMesh(core_axis_name="c", subcore_axis_name="s")

    BN = 2000
    xq, xk, xv = pl.pallas_call(
        _tc_proj_body,
        grid=(N // BN,),
        in_specs=[pl.BlockSpec((BN, D), lambda i: (i, 0))] +
                 [pl.BlockSpec((D, D), lambda i: (0, 0))] * 3,
        out_specs=[pl.BlockSpec((BN, D), lambda i: (i, 0))] * 3,
        out_shape=[jax.ShapeDtypeStruct((N, D), f32)] * 3,
    )(x, Wq, Wk, Wv)

    BE = 4000
    wea_pad = jnp.zeros((D, L), f32).at[:, :8].set(We_attn)
    ew = pl.pallas_call(
        _tc_ew_body,
        grid=(E // BE,),
        in_specs=[pl.BlockSpec((BE, D), lambda i: (i, 0)),
                  pl.BlockSpec((D, L), lambda i: (0, 0))],
        out_specs=pl.BlockSpec((BE, L), lambda i: (i, 0)),
        out_shape=jax.ShapeDtypeStruct((E, L), f32),
    )(e, wea_pad)

    CHA = 64
    attn_kernel = pl.kernel(
        functools.partial(_sc_attn_body, E // CHA, CHA, N),
        out_type=[jax.ShapeDtypeStruct((E, L), f32),
                  jax.ShapeDtypeStruct((N, D), f32),
                  jax.ShapeDtypeStruct((N, D), f32)],
        mesh=mesh,
        compiler_params=pltpu.CompilerParams(needs_layout_passes=False),
        scratch_types=[
            pltpu.VMEM((CHA,), jnp.int32),
            pltpu.VMEM((CHA,), jnp.int32),
            pltpu.VMEM((CHA, D), f32),
            pltpu.VMEM((CHA, D), f32),
            pltpu.VMEM((CHA, L), f32),
            pltpu.VMEM((CHA, L), f32),
            pltpu.VMEM((CHA, D), f32),
            pltpu.VMEM_SHARED((N, D), f32),
            pltpu.SemaphoreType.DMA,
        ],
    )
    ei_src = edge_index[0]
    ei_dst = edge_index[1]
    aexp, den0, den1 = attn_kernel(xq, xk, ew, ei_src, ei_dst)

    agg_kernel = pl.kernel(
        functools.partial(_sc_agg_body, nch, N),
        out_type=[jax.ShapeDtypeStruct((N, D), f32),
                  jax.ShapeDtypeStruct((N, D), f32)],
        mesh=mesh,
        compiler_params=pltpu.CompilerParams(needs_layout_passes=False),
        scratch_types=[
            pltpu.VMEM((CH,), jnp.int32),
            pltpu.VMEM((CH,), jnp.int32),
            pltpu.VMEM((CH, D), f32),
            pltpu.VMEM((CH, D), f32),
            pltpu.VMEM((CH, L), f32),
            pltpu.VMEM_SHARED((N, D), f32),
            pltpu.SemaphoreType.DMA,
        ],
    )
    o0, o1 = agg_kernel(xv, aexp, ei_src, ei_dst)

    weu1 = Weu[:D]
    weu2 = Weu[D:]
    pexp = jnp.zeros((D, D), f32)
    for _h in range(8):
        pexp = pexp.at[_h, _h * 16:(_h + 1) * 16].set(1.0)
    vec = lambda: pl.BlockSpec((D,), lambda i: (0,))
    x_out, a_nodes, b_nodes = pl.pallas_call(
        _tc_node_body,
        grid=(N // BN,),
        in_specs=[pl.BlockSpec((BN, D), lambda i: (i, 0))] * 2 +
                 [pl.BlockSpec((BN, D), lambda i: (i, 0))] * 2 +
                 [pl.BlockSpec((D, D), lambda i: (0, 0))] +
                 [pl.BlockSpec((BN, D), lambda i: (i, 0))] +
                 [pl.BlockSpec((D, D), lambda i: (0, 0)), vec(), vec(), vec(),
                  pl.BlockSpec((D, D), lambda i: (0, 0)),
                  pl.BlockSpec((D, D), lambda i: (0, 0)), vec(),
                  pl.BlockSpec((D, 2 * D), lambda i: (0, 0)),
                  pl.BlockSpec((2 * D,), lambda i: (0,)),
                  pl.BlockSpec((2 * D, D), lambda i: (0, 0)),
                  vec(), vec(), vec()],
        out_specs=[pl.BlockSpec((BN, D), lambda i: (i, 0))] * 3,
        out_shape=[jax.ShapeDtypeStruct((N, D), f32)] * 3,
    )(o0, o1, den0, den1, pexp, x, Wo, bo, g1, b1, weu1, weu2, beu,
      Wf1, bf1, Wf2, bf2, g2, b2)

    eg_kernel = pl.kernel(
        functools.partial(_sc_eg_body, nch),
        out_type=jax.ShapeDtypeStruct((E, D), f32),
        mesh=mesh,
        compiler_params=pltpu.CompilerParams(needs_layout_passes=False),
        scratch_types=[
            pltpu.VMEM((CH,), jnp.int32),
            pltpu.VMEM((CH,), jnp.int32),
            pltpu.VMEM((CH, D), f32),
            pltpu.VMEM((CH, D), f32),
            pltpu.SemaphoreType.DMA,
        ],
    )
    g_edges = eg_kernel(a_nodes, b_nodes, ei_src, ei_dst)

    e_out = pl.pallas_call(
        _tc_eout_body,
        grid=(E // BE,),
        in_specs=[pl.BlockSpec((BE, D), lambda i: (i, 0)),
                  pl.BlockSpec((BE, D), lambda i: (i, 0)), vec(), vec()],
        out_specs=pl.BlockSpec((BE, D), lambda i: (i, 0)),
        out_shape=jax.ShapeDtypeStruct((E, D), f32),
    )(e, g_edges, ge_w, be_w)

    return (x_out, e_out)


# speedup vs baseline: 3.5369x; 1.0182x over previous
import functools
import math

import jax
import jax.numpy as jnp
from jax import lax
from jax.experimental import pallas as pl
from jax.experimental.pallas import tpu as pltpu
from jax.experimental.pallas import tpu_sc as plsc

NC = 2
NS = 16
NW = NC * NS
L = 16
CH = 128


def _tc_proj_body(x_ref, wq_ref, wk_ref, wv_ref, xq_ref, xk_ref, xv_ref):
    xb = x_ref[...]
    xq_ref[...] = jnp.dot(xb, wq_ref[...], preferred_element_type=jnp.float32)
    xk_ref[...] = jnp.dot(xb, wk_ref[...], preferred_element_type=jnp.float32)
    xv_ref[...] = jnp.dot(xb, wv_ref[...], preferred_element_type=jnp.float32)


def _tc_ew_body(e_ref, w_ref, ew_ref):
    ew_ref[...] = jnp.dot(e_ref[...], w_ref[...],
                          preferred_element_type=jnp.float32)


def _layer_norm(v, g, b, eps=1e-5):
    m = jnp.mean(v, axis=-1, keepdims=True)
    var = jnp.mean((v - m) * (v - m), axis=-1, keepdims=True)
    return (v - m) / jnp.sqrt(var + eps) * g + b


def _tc_node_body(o0_ref, o1_ref, d0_ref, d1_ref, p_ref, x_ref, wo_ref,
                  bo_ref, g1_ref, b1_ref,
                  weu1_ref, weu2_ref, beu_ref, wf1_ref, bf1_ref, wf2_ref,
                  bf2_ref, g2_ref, b2_ref, xout_ref, a_ref, bbuf_ref):
    den = jnp.dot(d0_ref[...] + d1_ref[...], p_ref[...],
                  preferred_element_type=jnp.float32) + 1e-9
    agg = (o0_ref[...] + o1_ref[...]) / den
    out = jnp.dot(agg, wo_ref[...],
                  preferred_element_type=jnp.float32) + bo_ref[...]
    x1 = _layer_norm(x_ref[...] + out, g1_ref[...], b1_ref[...])
    a_ref[...] = jnp.dot(x1, weu1_ref[...], preferred_element_type=jnp.float32)
    bbuf_ref[...] = jnp.dot(x1, weu2_ref[...],
                            preferred_element_type=jnp.float32) + beu_ref[...]
    h = jnp.dot(x1, wf1_ref[...], preferred_element_type=jnp.float32) + bf1_ref[...]
    h = 0.5 * h * (1.0 + lax.erf(h * (1.0 / math.sqrt(2.0))))
    h = jnp.dot(h, wf2_ref[...], preferred_element_type=jnp.float32) + bf2_ref[...]
    xout_ref[...] = _layer_norm(x1 + h, g2_ref[...], b2_ref[...])


def _tc_eout_body(e_ref, g_ref, gw_ref, bw_ref, eout_ref):
    eout_ref[...] = _layer_norm(e_ref[...] + g_ref[...], gw_ref[...], bw_ref[...])


def _wid(cid, sid):
    return sid * NC + cid


def _node_split(n_nodes):
    rps8 = (n_nodes // NS) // 8 * 8
    tail = n_nodes - rps8 * NS
    return rps8, tail


def _sliced_copy(src_ref, dst_ref, sid, rps8, tail):
    nfull, rem = rps8 // CH, rps8 % CH

    @pl.loop(0, nfull)
    def _cp(t):
        rs = pl.ds(sid * rps8 + t * CH, CH)
        pltpu.sync_copy(src_ref.at[rs], dst_ref.at[rs])
    if rem:
        rs = pl.ds(sid * rps8 + nfull * CH, rem)
        pltpu.sync_copy(src_ref.at[rs], dst_ref.at[rs])
    if tail:
        @pl.when(sid == NS - 1)
        def _():
            ts = pl.ds(NS * rps8, tail)
            pltpu.sync_copy(src_ref.at[ts], dst_ref.at[ts])


def _zero_fill(zbuf_ref, dst_ref, sid, rps8, tail):
    zlen = zbuf_ref.shape[0]
    nfull, rem = rps8 // zlen, rps8 % zlen

    @pl.loop(0, nfull)
    def _cp(t):
        pltpu.sync_copy(zbuf_ref,
                        dst_ref.at[pl.ds(sid * rps8 + t * zlen, zlen)])
    if rem:
        pltpu.sync_copy(zbuf_ref.at[pl.ds(0, rem)],
                        dst_ref.at[pl.ds(sid * rps8 + nfull * zlen, rem)])
    if tail:
        @pl.when(sid == NS - 1)
        def _():
            pltpu.sync_copy(zbuf_ref.at[pl.ds(0, tail)],
                            dst_ref.at[pl.ds(NS * rps8, tail)])


def _sc_attn_body(nch, ch, n_nodes, xq_hbm, xk_hbm, ew_hbm, src_hbm, dst_hbm,
                  aexp_hbm, d0_hbm, d1_hbm,
                  src_v, dst_v, q_v, k_v, ew_v, aexp_v, pad_v, dsh, sem):
    cid = lax.axis_index("c")
    sid = lax.axis_index("s")
    w = _wid(cid, sid)
    rps8, tail = _node_split(n_nodes)

    @pl.loop(0, ch)
    def _zb(i):
        for h in range(8):
            pad_v[i, pl.ds(h * 16, 16)] = jnp.zeros((16,), jnp.float32)
    _zero_fill(pad_v, dsh, sid, rps8, tail)
    plsc.subcore_barrier()

    lanes = lax.iota(jnp.int32, L)
    swap8 = jnp.bitwise_xor(lanes, 8)
    n_iters = (nch + NW - 1) // NW

    @pl.loop(0, n_iters)
    def _chunk(j):
        c = w + NW * j

        @pl.when(c < nch)
        def _():
            base = c * ch
            pltpu.sync_copy(src_hbm.at[pl.ds(base, ch)], src_v)
            pltpu.sync_copy(dst_hbm.at[pl.ds(base, ch)], dst_v)
            dq = pltpu.async_copy(xq_hbm.at[dst_v], q_v, sem)
            dk = pltpu.async_copy(xk_hbm.at[src_v], k_v, sem)
            pltpu.sync_copy(ew_hbm.at[pl.ds(base, ch)], ew_v)
            dq.wait()
            dk.wait()

            @pl.loop(0, ch)
            def _edge(i):
                acc = q_v[i, pl.ds(0, 16)] * k_v[i, pl.ds(0, 16)]
                for r in range(1, 8):
                    sl = pl.ds(r * 16, 16)
                    acc = acc + q_v[i, sl] * k_v[i, sl]
                av = acc + jnp.take(acc, swap8)
                a = av * 0.25 + ew_v[i, :]
                row = jnp.where(lanes < 8, jnp.exp(a), 0.0)
                aexp_v[i, :] = row
                pad_v[i, pl.ds(0, L)] = row

            pltpu.sync_copy(pad_v, dsh.at[dst_v], add=True)
            pltpu.sync_copy(aexp_v, aexp_hbm.at[pl.ds(base, ch)])

    plsc.subcore_barrier()

    @pl.when(cid == 0)
    def _():
        _sliced_copy(dsh, d0_hbm, sid, rps8, tail)

    @pl.when(cid == 1)
    def _():
        _sliced_copy(dsh, d1_hbm, sid, rps8, tail)


def _sc_agg_body(nch, n_nodes, xv_hbm, aexp_hbm, src_hbm,
                 dst_hbm, o0_hbm, o1_hbm,
                 src_v, dst_v, v_v, sc_v, aexp_v, osh, sem):
    cid = lax.axis_index("c")
    sid = lax.axis_index("s")
    w = _wid(cid, sid)
    rps8, tail = _node_split(n_nodes)
    swap8 = jnp.bitwise_xor(lax.iota(jnp.int32, L), 8)

    @pl.loop(0, CH)
    def _zb(i):
        for h in range(8):
            sc_v[i, pl.ds(h * 16, 16)] = jnp.zeros((16,), jnp.float32)
    _zero_fill(sc_v, osh, sid, rps8, tail)
    plsc.subcore_barrier()

    n_iters = (nch + NW - 1) // NW

    @pl.loop(0, n_iters)
    def _chunk(j):
        c = w + NW * j

        @pl.when(c < nch)
        def _():
            base = c * CH
            pltpu.sync_copy(src_hbm.at[pl.ds(base, CH)], src_v)
            pltpu.sync_copy(dst_hbm.at[pl.ds(base, CH)], dst_v)
            dv = pltpu.async_copy(xv_hbm.at[src_v], v_v, sem)
            pltpu.sync_copy(aexp_hbm.at[pl.ds(base, CH)], aexp_v)
            dv.wait()

            @pl.loop(0, CH)
            def _scale(i):
                nr = aexp_v[i, :]
                mult = nr + jnp.take(nr, swap8)
                for r in range(8):
                    sl = pl.ds(r * 16, 16)
                    sc_v[i, sl] = v_v[i, sl] * mult

            pltpu.sync_copy(sc_v, osh.at[dst_v], add=True)

    plsc.subcore_barrier()

    @pl.when(cid == 0)
    def _():
        _sliced_copy(osh, o0_hbm, sid, rps8, tail)

    @pl.when(cid == 1)
    def _():
        _sliced_copy(osh, o1_hbm, sid, rps8, tail)


def _sc_eg_body(nch, a_hbm, b_hbm, src_hbm, dst_hbm, g_hbm,
                src_v, dst_v, a_v, b_v, sem):
    cid = lax.axis_index("c")
    sid = lax.axis_index("s")
    w = _wid(cid, sid)
    n_iters = (nch + NW - 1) // NW

    @pl.loop(0, n_iters)
    def _chunk(j):
        c = w + NW * j

        @pl.when(c < nch)
        def _():
            base = c * CH
            pltpu.sync_copy(src_hbm.at[pl.ds(base, CH)], src_v)
            pltpu.sync_copy(dst_hbm.at[pl.ds(base, CH)], dst_v)
            da = pltpu.async_copy(a_hbm.at[src_v], a_v, sem)
            db = pltpu.async_copy(b_hbm.at[dst_v], b_v, sem)
            da.wait()
            db.wait()

            @pl.loop(0, CH)
            def _edge(i):
                for h in range(8):
                    sl = pl.ds(h * 16, 16)
                    a_v[i, sl] = a_v[i, sl] + b_v[i, sl]

            pltpu.sync_copy(a_v, g_hbm.at[pl.ds(base, CH)])


def kernel(x, e, edge_index, Wq, Wk, Wv, We_attn, Wo, bo, Weu, beu,
           g1, b1, ge_w, be_w, g2, b2, Wf1, bf1, Wf2, bf2):
    N, D = x.shape
    E = e.shape[0]
    nch = E // CH
    f32 = jnp.float32
    mesh = plsc.VectorSubcoreMesh(core_axis_name="c", subcore_axis_name="s")

    perm = jnp.asarray([(j % 8) * 16 + j // 8 for j in range(D)], jnp.int32)
    BN = 2000
    xq, xk, xv = pl.pallas_call(
        _tc_proj_body,
        grid=(N // BN,),
        in_specs=[pl.BlockSpec((BN, D), lambda i: (i, 0))] +
                 [pl.BlockSpec((D, D), lambda i: (0, 0))] * 3,
        out_specs=[pl.BlockSpec((BN, D), lambda i: (i, 0))] * 3,
        out_shape=[jax.ShapeDtypeStruct((N, D), f32)] * 3,
    )(x, Wq[:, perm], Wk[:, perm], Wv[:, perm])

    BE = 4000
    wea_pad = jnp.zeros((D, L), f32).at[:, :8].set(We_attn)
    ew = pl.pallas_call(
        _tc_ew_body,
        grid=(E // BE,),
        in_specs=[pl.BlockSpec((BE, D), lambda i: (i, 0)),
                  pl.BlockSpec((D, L), lambda i: (0, 0))],
        out_specs=pl.BlockSpec((BE, L), lambda i: (i, 0)),
        out_shape=jax.ShapeDtypeStruct((E, L), f32),
    )(e, wea_pad)

    CHA = 64
    attn_kernel = pl.kernel(
        functools.partial(_sc_attn_body, E // CHA, CHA, N),
        out_type=[jax.ShapeDtypeStruct((E, L), f32),
                  jax.ShapeDtypeStruct((N, D), f32),
                  jax.ShapeDtypeStruct((N, D), f32)],
        mesh=mesh,
        compiler_params=pltpu.CompilerParams(needs_layout_passes=False),
        scratch_types=[
            pltpu.VMEM((CHA,), jnp.int32),
            pltpu.VMEM((CHA,), jnp.int32),
            pltpu.VMEM((CHA, D), f32),
            pltpu.VMEM((CHA, D), f32),
            pltpu.VMEM((CHA, L), f32),
            pltpu.VMEM((CHA, L), f32),
            pltpu.VMEM((CHA, D), f32),
            pltpu.VMEM_SHARED((N, D), f32),
            pltpu.SemaphoreType.DMA,
        ],
    )
    ei_src = edge_index[0]
    ei_dst = edge_index[1]
    aexp, den0, den1 = attn_kernel(xq, xk, ew, ei_src, ei_dst)

    agg_kernel = pl.kernel(
        functools.partial(_sc_agg_body, nch, N),
        out_type=[jax.ShapeDtypeStruct((N, D), f32),
                  jax.ShapeDtypeStruct((N, D), f32)],
        mesh=mesh,
        compiler_params=pltpu.CompilerParams(needs_layout_passes=False),
        scratch_types=[
            pltpu.VMEM((CH,), jnp.int32),
            pltpu.VMEM((CH,), jnp.int32),
            pltpu.VMEM((CH, D), f32),
            pltpu.VMEM((CH, D), f32),
            pltpu.VMEM((CH, L), f32),
            pltpu.VMEM_SHARED((N, D), f32),
            pltpu.SemaphoreType.DMA,
        ],
    )
    o0, o1 = agg_kernel(xv, aexp, ei_src, ei_dst)

    weu1 = Weu[:D]
    weu2 = Weu[D:]
    pexp = jnp.zeros((D, D), f32)
    for _h in range(8):
        pexp = pexp.at[_h, _h::8].set(1.0)
    vec = lambda: pl.BlockSpec((D,), lambda i: (0,))
    x_out, a_nodes, b_nodes = pl.pallas_call(
        _tc_node_body,
        grid=(N // BN,),
        in_specs=[pl.BlockSpec((BN, D), lambda i: (i, 0))] * 2 +
                 [pl.BlockSpec((BN, D), lambda i: (i, 0))] * 2 +
                 [pl.BlockSpec((D, D), lambda i: (0, 0))] +
                 [pl.BlockSpec((BN, D), lambda i: (i, 0))] +
                 [pl.BlockSpec((D, D), lambda i: (0, 0)), vec(), vec(), vec(),
                  pl.BlockSpec((D, D), lambda i: (0, 0)),
                  pl.BlockSpec((D, D), lambda i: (0, 0)), vec(),
                  pl.BlockSpec((D, 2 * D), lambda i: (0, 0)),
                  pl.BlockSpec((2 * D,), lambda i: (0,)),
                  pl.BlockSpec((2 * D, D), lambda i: (0, 0)),
                  vec(), vec(), vec()],
        out_specs=[pl.BlockSpec((BN, D), lambda i: (i, 0))] * 3,
        out_shape=[jax.ShapeDtypeStruct((N, D), f32)] * 3,
    )(o0, o1, den0, den1, pexp, x, Wo[perm, :], bo, g1, b1, weu1, weu2, beu,
      Wf1, bf1, Wf2, bf2, g2, b2)

    eg_kernel = pl.kernel(
        functools.partial(_sc_eg_body, nch),
        out_type=jax.ShapeDtypeStruct((E, D), f32),
        mesh=mesh,
        compiler_params=pltpu.CompilerParams(needs_layout_passes=False),
        scratch_types=[
            pltpu.VMEM((CH,), jnp.int32),
            pltpu.VMEM((CH,), jnp.int32),
            pltpu.VMEM((CH, D), f32),
            pltpu.VMEM((CH, D), f32),
            pltpu.SemaphoreType.DMA,
        ],
    )
    g_edges = eg_kernel(a_nodes, b_nodes, ei_src, ei_dst)

    e_out = pl.pallas_call(
        _tc_eout_body,
        grid=(E // BE,),
        in_specs=[pl.BlockSpec((BE, D), lambda i: (i, 0)),
                  pl.BlockSpec((BE, D), lambda i: (i, 0)), vec(), vec()],
        out_specs=pl.BlockSpec((BE, D), lambda i: (i, 0)),
        out_shape=jax.ShapeDtypeStruct((E, D), f32),
    )(e, g_edges, ge_w, be_w)

    return (x_out, e_out)
